# baseline (speedup 1.0000x reference)
import functools

import jax
import jax.numpy as jnp
from jax import lax
from jax.experimental import pallas as pl
from jax.experimental.pallas import tpu as pltpu
from jax.experimental.pallas import tpu_sc as plsc

D = 64
NC = 2
NS = 16
NW = NC * NS
CH = 512
NBUF = 2
IDX_PER_STREAM = 128
NSTREAM = CH // IDX_PER_STREAM
SCALE = float(D) ** 0.5


def _sc_embedding_gather(table, idx_flat, B):
    b_per_w = B // NW
    nch = b_per_w // CH
    mesh = plsc.VectorSubcoreMesh(core_axis_name="c", subcore_axis_name="s")

    @functools.partial(
        pl.kernel,
        out_type=jax.ShapeDtypeStruct((B, D), jnp.float32),
        mesh=mesh,
        compiler_params=pltpu.CompilerParams(use_tc_tiling_on_sc=False),
        scratch_types=[
            pltpu.VMEM((b_per_w,), jnp.int32),
            pltpu.VMEM((CH, D), jnp.float32),
            pltpu.VMEM((CH, D), jnp.float32),
            pltpu.SemaphoreType.DMA,
            pltpu.SemaphoreType.DMA,
            pltpu.SemaphoreType.DMA,
            pltpu.SemaphoreType.DMA,
        ],
    )
    def k(table_hbm, idx_hbm, out_hbm, idx_v, rows0, rows1, g0, g1, s0, s1):
        rows = (rows0, rows1)
        gsem = (g0, g1)
        ssem = (s0, s1)
        wid = lax.axis_index("s") * NC + lax.axis_index("c")
        base = wid * b_per_w

        pltpu.sync_copy(idx_hbm.at[pl.ds(base, b_per_w)], idx_v)

        def fire_gathers(g, b):
            off = g * CH
            for j in range(NSTREAM):
                pltpu.async_copy(
                    table_hbm.at[idx_v.at[pl.ds(off + j * IDX_PER_STREAM,
                                                IDX_PER_STREAM)]],
                    rows[b].at[pl.ds(j * IDX_PER_STREAM, IDX_PER_STREAM)],
                    gsem[b],
                )

        def wait_gathers(b):
            pltpu.make_async_copy(
                table_hbm.at[pl.ds(0, CH)], rows[b], gsem[b]).wait()

        def compute(g, b):
            rb = rows[b]
            chunk_off = g * CH

            def tbody(t, carry):
                iv = idx_v[pl.ds(chunk_off + t * 16, 16)]
                sv = jnp.where(iv == 0, 0.0, SCALE).astype(jnp.float32)
                dnums = lax.GatherDimensionNumbers(
                    offset_dims=(), collapsed_slice_dims=(0,),
                    start_index_map=(0,))
                for l in range(16):
                    spl = lax.gather(
                        sv, jnp.full((16, 1), l, jnp.int32), dnums,
                        slice_sizes=(1,),
                        mode=lax.GatherScatterMode.PROMISE_IN_BOUNDS)
                    r = t * 16 + l
                    for q in range(D // 16):
                        sl = pl.ds(q * 16, 16)
                        rb[r, sl] = rb[r, sl] * spl
                return carry

            lax.fori_loop(0, CH // 16, tbody, 0)

        def start_store(g, b):
            pltpu.async_copy(
                rows[b], out_hbm.at[pl.ds(base + g * CH, CH)], ssem[b])

        def wait_store(b):
            pltpu.make_async_copy(
                rows[b], out_hbm.at[pl.ds(0, CH)], ssem[b]).wait()

        for b in range(NBUF):
            fire_gathers(b, b)

        def outer(o, carry):
            for b in range(NBUF):
                g = o * NBUF + b
                wait_gathers(b)
                compute(g, b)
                start_store(g, b)
                wait_store(b)

                @pl.when(g + NBUF < nch)
                def _():
                    fire_gathers(g + NBUF, b)
            return carry

        lax.fori_loop(0, nch // NBUF, outer, 0)

    return k(table, idx_flat)


def kernel(inputs, shared_weights):
    bsz, seq = inputs.shape
    B = bsz * seq
    idx_flat = inputs.astype(jnp.int32).reshape(B)
    out = _sc_embedding_gather(shared_weights, idx_flat, B)
    return out.reshape(bsz, seq, D)


# speedup vs baseline: 1.1294x; 1.1294x over previous
import functools

import jax
import jax.numpy as jnp
from jax import lax
from jax.experimental import pallas as pl
from jax.experimental.pallas import tpu as pltpu
from jax.experimental.pallas import tpu_sc as plsc

D = 64
NC = 2
NS = 16
NW = NC * NS
CB = 256
NBUF = 4
IDX_PER_STREAM = 128
NSTREAM = 2 * CB // IDX_PER_STREAM
SCALE = float(D) ** 0.5


def _sc_embedding_gather(t32, idx_flat, B):
    b_per_w = B // NW
    nch = b_per_w // CB
    mesh = plsc.VectorSubcoreMesh(core_axis_name="c", subcore_axis_name="s")

    @functools.partial(
        pl.kernel,
        out_type=jax.ShapeDtypeStruct((2 * B, 32), jnp.float32),
        mesh=mesh,
        compiler_params=pltpu.CompilerParams(use_tc_tiling_on_sc=False),
        scratch_types=(
            [pltpu.VMEM((b_per_w,), jnp.int32)]
            + [pltpu.VMEM((2 * CB,), jnp.int32) for _ in range(NBUF)]
            + [pltpu.VMEM((2 * CB, 32), jnp.float32) for _ in range(NBUF)]
            + [pltpu.SemaphoreType.DMA for _ in range(2 * NBUF)]
        ),
    )
    def k(t32_hbm, idx_hbm, out_hbm, idx_v, *bufs):
        h = bufs[:NBUF]
        r = bufs[NBUF:2 * NBUF]
        gsem = bufs[2 * NBUF:3 * NBUF]
        ssem = bufs[3 * NBUF:]
        wid = lax.axis_index("s") * NC + lax.axis_index("c")
        base = wid * b_per_w

        pltpu.sync_copy(idx_hbm.at[pl.ds(base, b_per_w)], idx_v)

        lane = lax.broadcasted_iota(jnp.int32, (16,), 0)
        alt = lane & 1
        perm_lo = jnp.expand_dims(lane >> 1, 1)
        perm_hi = jnp.expand_dims(8 + (lane >> 1), 1)
        dnums = lax.GatherDimensionNumbers(
            offset_dims=(), collapsed_slice_dims=(0,), start_index_map=(0,))

        def interleave(iv, perm):
            rep = lax.gather(iv, perm, dnums, slice_sizes=(1,),
                             mode=lax.GatherScatterMode.PROMISE_IN_BOUNDS)
            return rep + rep + alt

        def fire_gathers(g, b):
            off = g * CB
            hb = h[b]

            def hsetup(t, carry):
                iv = idx_v[pl.ds(off + t * 16, 16)]
                hb[pl.ds(t * 32, 16)] = interleave(iv, perm_lo)
                hb[pl.ds(t * 32 + 16, 16)] = interleave(iv, perm_hi)
                return carry

            lax.fori_loop(0, CB // 16, hsetup, 0)
            for j in range(NSTREAM):
                pltpu.async_copy(
                    t32_hbm.at[hb.at[pl.ds(j * IDX_PER_STREAM,
                                           IDX_PER_STREAM)]],
                    r[b].at[pl.ds(j * IDX_PER_STREAM, IDX_PER_STREAM)],
                    gsem[b],
                )

        def wait_gathers(b):
            pltpu.make_async_copy(
                t32_hbm.at[pl.ds(0, 2 * CB)], r[b], gsem[b]).wait()

        def start_store(g, b):
            pltpu.async_copy(
                r[b], out_hbm.at[pl.ds((base + g * CB) * 2, 2 * CB)],
                ssem[b])

        def wait_store(b):
            pltpu.make_async_copy(
                r[b], out_hbm.at[pl.ds(0, 2 * CB)], ssem[b]).wait()

        for b in range(NBUF):
            fire_gathers(b, b)

        def outer(o, carry):
            for b in range(NBUF):
                g = o * NBUF + b
                wait_gathers(b)
                start_store(g, b)

                @pl.when(g + NBUF < nch)
                def _():
                    wait_store(b)
                    fire_gathers(g + NBUF, b)
            return carry

        lax.fori_loop(0, nch // NBUF, outer, 0)
        for b in range(NBUF):
            wait_store(b)

    return k(t32, idx_flat)


def kernel(inputs, shared_weights):
    bsz, seq = inputs.shape
    B = bsz * seq
    vocab = shared_weights.shape[0]
    idx_flat = inputs.astype(jnp.int32).reshape(B)
    rowid = lax.broadcasted_iota(jnp.int32, (vocab, 1), 0)
    stable = jnp.where(rowid == 0, 0.0, shared_weights * SCALE)
    t32 = stable.reshape(2 * vocab, D // 2)
    out = _sc_embedding_gather(t32, idx_flat, B)
    return out.reshape(bsz, seq, D)
